# baseline (speedup 1.0000x reference)
import jax
import jax.numpy as jnp
from jax import lax
from jax.experimental import pallas as pl
from jax.experimental.pallas import tpu as pltpu
from jax.experimental.pallas import tpu_sc as plsc

_F = 128
_H = 128
_W = 128
_NW = 16
_CPH = _F // _NW


def _sc_body(col_hbm, row_hbm, out_hbm, tab_c, tab_r, vec_v, planes, sem):
    nc = 1
    wid = lax.axis_index("s") * nc + lax.axis_index("c")

    pltpu.sync_copy(col_hbm.at[pl.ds(0, _W)], tab_c)
    pltpu.sync_copy(row_hbm.at[pl.ds(0, _H)], tab_r)

    lane = lax.iota(jnp.int32, 16)
    fbase = wid * _CPH

    copies = [None, None]
    for k in range(2 * _CPH):
        is_col = k < _CPH
        kk = k if is_col else k - _CPH
        f = fbase + kk
        c = f if is_col else _F + f
        tab = tab_c if is_col else tab_r
        pbuf = planes.at[k % 2]

        fvec = jnp.full((16,), f, dtype=jnp.int32)
        for m in range(_H // 16):
            vec_v[pl.ds(16 * m, 16)] = plsc.load_gather(tab, [lane + 16 * m, fvec])

        if copies[k % 2] is not None:
            copies[k % 2].wait()

        if is_col:
            chunks = [vec_v[pl.ds(16 * m, 16)] for m in range(_W // 16)]

            def fill_col(g, carry):
                for l in range(16):
                    i = g * 16 + l
                    for m in range(_W // 16):
                        pbuf[i, pl.ds(16 * m, 16)] = chunks[m]
                return carry

            lax.fori_loop(0, _H // 16, fill_col, 0)
        else:
            def fill_row(g, carry):
                splats = [
                    plsc.load_gather(
                        vec_v, [jnp.full((16,), g * 16 + l, dtype=jnp.int32)]
                    )
                    for l in range(16)
                ]
                for l in range(16):
                    i = g * 16 + l
                    for m in range(_W // 16):
                        pbuf[i, pl.ds(16 * m, 16)] = splats[l]
                return carry

            lax.fori_loop(0, _H // 16, fill_row, 0)

        cp = pltpu.make_async_copy(pbuf, out_hbm.at[c], sem.at[k % 2])
        cp.start()
        copies[k % 2] = cp

    for cp in copies:
        if cp is not None:
            cp.wait()


def kernel(image_tensor, row_embed, col_embed):
    h, w = image_tensor.shape[-2], image_tensor.shape[-1]
    F = row_embed.shape[1]
    mesh = plsc.VectorSubcoreMesh(core_axis_name="c", subcore_axis_name="s", num_cores=1)
    run = pl.kernel(
        _sc_body,
        out_type=jax.ShapeDtypeStruct((2 * F, h, w), jnp.float32),
        mesh=mesh,
        compiler_params=pltpu.CompilerParams(needs_layout_passes=False),
        scratch_types=[
            pltpu.VMEM((w, F), jnp.float32),
            pltpu.VMEM((h, F), jnp.float32),
            pltpu.VMEM((h,), jnp.float32),
            pltpu.VMEM((2, h, w), jnp.float32),
            pltpu.SemaphoreType.DMA((2,)),
        ],
    )
    out = run(col_embed, row_embed)
    return out[None]


# speedup vs baseline: 4.4836x; 4.4836x over previous
import jax
import jax.numpy as jnp
from jax.experimental import pallas as pl
from jax.experimental.pallas import tpu as pltpu

_BC = 128


def _pos_kernel(col_ref, row_ref, out_ref):
    bc, h, w = out_ref.shape
    nb_half = pl.num_programs(0) // 2
    b = pl.program_id(0)
    for k in range(2 * nb_half):
        @pl.when(b == k)
        def _(k=k):
            if k < nb_half:
                slab = col_ref[:].T[k * bc:(k + 1) * bc, :]
                out_ref[...] = jnp.broadcast_to(slab[:, None, :], (bc, h, w))
            else:
                kk = k - nb_half
                slab = row_ref[:].T[kk * bc:(kk + 1) * bc, :]
                out_ref[...] = jnp.broadcast_to(slab[:, :, None], (bc, h, w))


def kernel(image_tensor, row_embed, col_embed):
    h, w = image_tensor.shape[-2], image_tensor.shape[-1]
    F = row_embed.shape[1]
    out = pl.pallas_call(
        _pos_kernel,
        grid=(2 * F // _BC,),
        in_specs=[
            pl.BlockSpec((w, F), lambda b: (0, 0)),
            pl.BlockSpec((h, F), lambda b: (0, 0)),
        ],
        out_specs=pl.BlockSpec((_BC, h, w), lambda b: (b, 0, 0)),
        out_shape=jax.ShapeDtypeStruct((2 * F, h, w), jnp.float32),
        compiler_params=pltpu.CompilerParams(dimension_semantics=("parallel",)),
    )(col_embed[:w], row_embed[:h])
    return out[None]


# speedup vs baseline: 6.1671x; 1.3755x over previous
import jax
import jax.numpy as jnp
from jax.experimental import pallas as pl
from jax.experimental.pallas import tpu as pltpu

_BC = 128


def _pos_kernel(col_ref, row_ref, out_ref):
    bc, h, w = out_ref.shape
    nb_half = pl.num_programs(0) // 2
    b = pl.program_id(0)
    for k in range(2 * nb_half):
        @pl.when(b == k)
        def _(k=k):
            if k < nb_half:
                slab = col_ref[0:w, :].T[k * bc:(k + 1) * bc, :]
                out_ref[...] = jnp.broadcast_to(slab[:, None, :], (bc, h, w))
            else:
                kk = k - nb_half
                slab = row_ref[0:h, :].T[kk * bc:(kk + 1) * bc, :]
                out_ref[...] = jnp.broadcast_to(slab[:, :, None], (bc, h, w))


def kernel(image_tensor, row_embed, col_embed):
    h, w = image_tensor.shape[-2], image_tensor.shape[-1]
    F = row_embed.shape[1]
    n_emb = row_embed.shape[0]
    out = pl.pallas_call(
        _pos_kernel,
        grid=(2 * F // _BC,),
        in_specs=[
            pl.BlockSpec((n_emb, F), lambda b: (0, 0)),
            pl.BlockSpec((n_emb, F), lambda b: (0, 0)),
        ],
        out_specs=pl.BlockSpec((_BC, h, w), lambda b: (b, 0, 0)),
        out_shape=jax.ShapeDtypeStruct((2 * F, h, w), jnp.float32),
        compiler_params=pltpu.CompilerParams(dimension_semantics=("parallel",)),
    )(col_embed, row_embed)
    return out[None]
